# baseline (speedup 1.0000x reference)
import functools

import jax
import jax.numpy as jnp
from jax.experimental import pallas as pl

N_NODES = 50000
ROW_BLOCK = 1000


def _stage_mm_body(base_ref, aggv_ref, xbase_ref, aggx_ref, We_ref, be_ref,
                   W_ref, b_ref, eps_ref, v_out_ref, h_out_ref, stats_ref):
    i = pl.program_id(0)
    vin = base_ref[...] + aggv_ref[...]
    xin = (1.0 + eps_ref[0, 0]) * xbase_ref[...] + aggx_ref[...]
    v_out = jnp.dot(vin, We_ref[...], preferred_element_type=jnp.float32)
    v_out = jnp.maximum(v_out + be_ref[...], 0.0)
    h = jnp.dot(xin, W_ref[...], preferred_element_type=jnp.float32) + b_ref[...]
    v_out_ref[...] = v_out
    h_out_ref[...] = h
    s0 = jnp.sum(h, axis=0, keepdims=True)
    s1 = jnp.sum(h * h, axis=0, keepdims=True)
    s = jnp.concatenate([s0, s1], axis=0)

    @pl.when(i == 0)
    def _():
        stats_ref[...] = s

    @pl.when(i > 0)
    def _():
        stats_ref[...] += s


def _stage_mm(base, aggv, xbase, aggx, We, be, W, b, eps):
    n, din = base.shape
    dout = We.shape[1]
    grid = n // ROW_BLOCK
    row = lambda i: (i, 0)
    full = lambda i: (0, 0)
    return pl.pallas_call(
        _stage_mm_body,
        grid=(grid,),
        in_specs=[
            pl.BlockSpec((ROW_BLOCK, din), row),
            pl.BlockSpec((ROW_BLOCK, din), row),
            pl.BlockSpec((ROW_BLOCK, din), row),
            pl.BlockSpec((ROW_BLOCK, din), row),
            pl.BlockSpec((din, dout), full),
            pl.BlockSpec((1, dout), full),
            pl.BlockSpec((din, dout), full),
            pl.BlockSpec((1, dout), full),
            pl.BlockSpec((1, 1), full),
        ],
        out_specs=[
            pl.BlockSpec((ROW_BLOCK, dout), row),
            pl.BlockSpec((ROW_BLOCK, dout), row),
            pl.BlockSpec((2, dout), full),
        ],
        out_shape=[
            jax.ShapeDtypeStruct((n, dout), jnp.float32),
            jax.ShapeDtypeStruct((n, dout), jnp.float32),
            jax.ShapeDtypeStruct((2, dout), jnp.float32),
        ],
    )(base, aggv, xbase, aggx, We, be.reshape(1, -1), W, b.reshape(1, -1),
      eps.reshape(1, 1))


def _stage_norm_body(h_ref, stats_ref, g_ref, bt_ref, vres_ref, out_ref):
    inv_n = 1.0 / N_NODES
    m = stats_ref[0:1, :] * inv_n
    var = stats_ref[1:2, :] * inv_n - m * m
    inv = jax.lax.rsqrt(var + 1e-5)
    x = (h_ref[...] - m) * inv * g_ref[...] + bt_ref[...]
    out_ref[...] = jnp.maximum(x, 0.0) + vres_ref[...]


def _stage_norm(h, stats, g, bt, vres):
    n, d = h.shape
    grid = n // ROW_BLOCK
    row = lambda i: (i, 0)
    full = lambda i: (0, 0)
    return pl.pallas_call(
        _stage_norm_body,
        grid=(grid,),
        in_specs=[
            pl.BlockSpec((ROW_BLOCK, d), row),
            pl.BlockSpec((2, d), full),
            pl.BlockSpec((1, d), full),
            pl.BlockSpec((1, d), full),
            pl.BlockSpec((ROW_BLOCK, d), row),
        ],
        out_specs=pl.BlockSpec((ROW_BLOCK, d), row),
        out_shape=jax.ShapeDtypeStruct((n, d), jnp.float32),
    )(h, stats, g.reshape(1, -1), bt.reshape(1, -1), vres)


def kernel(v, edges, We1, be1, We2, be2, We3, be3,
           eps1, W1, b1, g1, bt1,
           eps2, W2, b2, g2, bt2,
           eps3, W3, b3, g3, bt3):
    src = edges[0]
    dst = edges[1]

    def agg(x):
        return jax.ops.segment_sum(x[src], dst, num_segments=N_NODES)

    v_p = jnp.pad(v, ((0, 0), (0, 128 - 86)))
    We1p = jnp.pad(We1, ((0, 128 - 86), (0, 0)))
    W1p = jnp.pad(W1, ((0, 128 - 86), (0, 0)))

    A0 = agg(v_p)
    v1, h1, s1 = _stage_mm(v_p, A0, v_p, A0, We1p, be1, W1p, b1, eps1)
    xv1 = _stage_norm(h1, s1, g1, bt1, v1)

    A1v = agg(v1)
    A1x = agg(xv1)
    v2, h2, s2 = _stage_mm(v1, A1v, xv1, A1x, We2, be2, W2, b2, eps2)
    xv2 = _stage_norm(h2, s2, g2, bt2, v2)

    A2v = agg(v2)
    A2x = agg(xv2)
    v3, h3, s3 = _stage_mm(v2, A2v, xv2, A2x, We3, be3, W3, b3, eps3)
    xv3 = _stage_norm(h3, s3, g3, bt3, v3)
    return xv3


# speedup vs baseline: 1.8635x; 1.8635x over previous
import functools

import jax
import jax.numpy as jnp
from jax import lax
from jax.experimental import pallas as pl
from jax.experimental.pallas import tpu as pltpu
from jax.experimental.pallas import tpu_sc as plsc

N_NODES = 50000
ROW_BLOCK = 1000


NC = 2
NS = 16
NPAD = 50176
SLICE = NPAD // NS
EB = 128
NBATCH = 800000 // EB
FULL = NBATCH // NS
EXTRA = NBATCH - FULL * NS


def _make_sc_agg(K):
    mesh = plsc.VectorSubcoreMesh(core_axis_name="c", subcore_axis_name="s",
                                  num_cores=NC, num_subcores=NS)

    @functools.partial(
        pl.kernel,
        out_type=jax.ShapeDtypeStruct((K, NPAD, 32), jnp.float32),
        mesh=mesh,
        scratch_types=[
            pltpu.VMEM((EB,), jnp.int32),
            pltpu.VMEM((EB,), jnp.int32),
            pltpu.VMEM((EB,), jnp.int32),
            pltpu.VMEM((EB,), jnp.int32),
            pltpu.VMEM((EB, 32), jnp.float32),
            pltpu.VMEM((EB, 32), jnp.float32),
            pltpu.VMEM_SHARED((NPAD, 32), jnp.float32),
            pltpu.SemaphoreType.DMA,
            pltpu.SemaphoreType.DMA,
        ],
        compiler_params=pltpu.CompilerParams(use_tc_tiling_on_sc=False),
    )
    def agg(x_hbm, src_hbm, dst_hbm, zeros_hbm, out_hbm,
            sidx_a, sidx_b, didx_a, didx_b, rows_a, rows_b, acc, sem_a, sem_b):
        c = lax.axis_index("c")
        s = lax.axis_index("s")
        base_batch = s * FULL

        def load_idx(batch, sidx, didx, chunk_off):
            off = batch * EB
            pltpu.sync_copy(src_hbm.at[pl.ds(off, EB)], sidx)
            pltpu.sync_copy(dst_hbm.at[pl.ds(off, EB)], didx)
            cvec = jnp.full((16,), chunk_off, jnp.int32)
            for j in range(EB // 16):
                sl = pl.ds(16 * j, 16)
                sidx[sl] = sidx[sl] + cvec

        def gather(sidx, rows, sem):
            return pltpu.make_async_copy(x_hbm.at[sidx], rows, sem)

        def scatter(rows, didx):
            pltpu.sync_copy(rows, acc.at[didx], add=True)

        for ci in range(K // NC):
            chunk = ci * NC + c
            chunk_off = chunk * NPAD
            pltpu.sync_copy(zeros_hbm, acc.at[pl.ds(s * SLICE, SLICE)])
            plsc.subcore_barrier()

            load_idx(base_batch, sidx_a, didx_a, chunk_off)
            gather(sidx_a, rows_a, sem_a).start()

            def pair(i, _):
                g = base_batch + 2 * i
                load_idx(g + 1, sidx_b, didx_b, chunk_off)
                gather(sidx_b, rows_b, sem_b).start()
                gather(sidx_a, rows_a, sem_a).wait()
                scatter(rows_a, didx_a)

                @pl.when(2 * i + 2 < FULL)
                def _():
                    load_idx(g + 2, sidx_a, didx_a, chunk_off)
                    gather(sidx_a, rows_a, sem_a).start()

                gather(sidx_b, rows_b, sem_b).wait()
                scatter(rows_b, didx_b)
                return None

            lax.fori_loop(0, FULL // 2, pair, None)

            @pl.when(s < EXTRA)
            def _():
                load_idx(NS * FULL + s, sidx_a, didx_a, chunk_off)
                cp = gather(sidx_a, rows_a, sem_a)
                cp.start()
                cp.wait()
                scatter(rows_a, didx_a)

            plsc.subcore_barrier()
            pltpu.sync_copy(acc.at[pl.ds(s * SLICE, SLICE)],
                            out_hbm.at[chunk, pl.ds(s * SLICE, SLICE)])

    return agg


def _sc_agg(x, src, dst):
    n, d = x.shape
    K = d // 32
    xp = jnp.pad(x, ((0, NPAD - n), (0, 0)))
    xt = xp.reshape(NPAD, K, 32).transpose(1, 0, 2).reshape(K * NPAD, 32)
    zeros = jnp.zeros((SLICE, 32), jnp.float32)
    outt = _make_sc_agg(K)(xt, src, dst, zeros)
    return outt.transpose(1, 0, 2).reshape(NPAD, d)[:n]


def _stage_mm_body(base_ref, aggv_ref, xbase_ref, aggx_ref, We_ref, be_ref,
                   W_ref, b_ref, eps_ref, v_out_ref, h_out_ref, stats_ref):
    i = pl.program_id(0)
    vin = base_ref[...] + aggv_ref[...]
    xin = (1.0 + eps_ref[0, 0]) * xbase_ref[...] + aggx_ref[...]
    v_out = jnp.dot(vin, We_ref[...], preferred_element_type=jnp.float32)
    v_out = jnp.maximum(v_out + be_ref[...], 0.0)
    h = jnp.dot(xin, W_ref[...], preferred_element_type=jnp.float32) + b_ref[...]
    v_out_ref[...] = v_out
    h_out_ref[...] = h
    s0 = jnp.sum(h, axis=0, keepdims=True)
    s1 = jnp.sum(h * h, axis=0, keepdims=True)
    s = jnp.concatenate([s0, s1], axis=0)

    @pl.when(i == 0)
    def _():
        stats_ref[...] = s

    @pl.when(i > 0)
    def _():
        stats_ref[...] += s


def _stage_mm(base, aggv, xbase, aggx, We, be, W, b, eps):
    n, din = base.shape
    dout = We.shape[1]
    grid = n // ROW_BLOCK
    row = lambda i: (i, 0)
    full = lambda i: (0, 0)
    return pl.pallas_call(
        _stage_mm_body,
        grid=(grid,),
        in_specs=[
            pl.BlockSpec((ROW_BLOCK, din), row),
            pl.BlockSpec((ROW_BLOCK, din), row),
            pl.BlockSpec((ROW_BLOCK, din), row),
            pl.BlockSpec((ROW_BLOCK, din), row),
            pl.BlockSpec((din, dout), full),
            pl.BlockSpec((1, dout), full),
            pl.BlockSpec((din, dout), full),
            pl.BlockSpec((1, dout), full),
            pl.BlockSpec((1, 1), full),
        ],
        out_specs=[
            pl.BlockSpec((ROW_BLOCK, dout), row),
            pl.BlockSpec((ROW_BLOCK, dout), row),
            pl.BlockSpec((2, dout), full),
        ],
        out_shape=[
            jax.ShapeDtypeStruct((n, dout), jnp.float32),
            jax.ShapeDtypeStruct((n, dout), jnp.float32),
            jax.ShapeDtypeStruct((2, dout), jnp.float32),
        ],
    )(base, aggv, xbase, aggx, We, be.reshape(1, -1), W, b.reshape(1, -1),
      eps.reshape(1, 1))


def _stage_norm_body(h_ref, stats_ref, g_ref, bt_ref, vres_ref, out_ref):
    inv_n = 1.0 / N_NODES
    m = stats_ref[0:1, :] * inv_n
    var = stats_ref[1:2, :] * inv_n - m * m
    inv = jax.lax.rsqrt(var + 1e-5)
    x = (h_ref[...] - m) * inv * g_ref[...] + bt_ref[...]
    out_ref[...] = jnp.maximum(x, 0.0) + vres_ref[...]


def _stage_norm(h, stats, g, bt, vres):
    n, d = h.shape
    grid = n // ROW_BLOCK
    row = lambda i: (i, 0)
    full = lambda i: (0, 0)
    return pl.pallas_call(
        _stage_norm_body,
        grid=(grid,),
        in_specs=[
            pl.BlockSpec((ROW_BLOCK, d), row),
            pl.BlockSpec((2, d), full),
            pl.BlockSpec((1, d), full),
            pl.BlockSpec((1, d), full),
            pl.BlockSpec((ROW_BLOCK, d), row),
        ],
        out_specs=pl.BlockSpec((ROW_BLOCK, d), row),
        out_shape=jax.ShapeDtypeStruct((n, d), jnp.float32),
    )(h, stats, g.reshape(1, -1), bt.reshape(1, -1), vres)


def kernel(v, edges, We1, be1, We2, be2, We3, be3,
           eps1, W1, b1, g1, bt1,
           eps2, W2, b2, g2, bt2,
           eps3, W3, b3, g3, bt3):
    src = edges[0]
    dst = edges[1]

    v_p = jnp.pad(v, ((0, 0), (0, 128 - 86)))
    We1p = jnp.pad(We1, ((0, 128 - 86), (0, 0)))
    W1p = jnp.pad(W1, ((0, 128 - 86), (0, 0)))

    A0 = _sc_agg(v_p, src, dst)
    v1, h1, s1 = _stage_mm(v_p, A0, v_p, A0, We1p, be1, W1p, b1, eps1)
    xv1 = _stage_norm(h1, s1, g1, bt1, v1)

    A1 = _sc_agg(jnp.concatenate([v1, xv1], axis=1), src, dst)
    A1v, A1x = A1[:, :128], A1[:, 128:]
    v2, h2, s2 = _stage_mm(v1, A1v, xv1, A1x, We2, be2, W2, b2, eps2)
    xv2 = _stage_norm(h2, s2, g2, bt2, v2)

    A2 = _sc_agg(jnp.concatenate([v2, xv2], axis=1), src, dst)
    A2v, A2x = A2[:, :256], A2[:, 256:]
    v3, h3, s3 = _stage_mm(v2, A2v, xv2, A2x, We3, be3, W3, b3, eps3)
    xv3 = _stage_norm(h3, s3, g3, bt3, v3)
    return xv3
